# baseline (speedup 1.0000x reference)
import jax
import jax.numpy as jnp
from jax import lax
from jax.experimental import pallas as pl
from jax.experimental.pallas import tpu as pltpu
from jax.experimental.pallas import tpu_sc as plsc

N = 10000
E = 160000
FEAT = 128
NRBF = 50
CUTOFF = 5.0

CH = 32
NCHUNK = E // CH
CHG = 128
NCHUNKG = E // CHG
NSUB = 16
NCORE = 2
NW = NSUB * NCORE
NHALF = N // 2

HB = NHALF + 8
NACC2 = 2 * NHALF + 16

ROWS_A = 624
TAIL_A = N - NSUB * ROWS_A
ROWS_B = 312
TAIL_B = NHALF - NSUB * ROWS_B


def _softplus(x):
    return jnp.maximum(x, 0.0) + jnp.log1p(jnp.exp(-jnp.abs(x)))


def _phi_body(h_ref, w1_ref, b1_ref, w2_ref, b2_ref, o_ref):
    h = h_ref[...].astype(jnp.bfloat16)
    z = jnp.dot(h, w1_ref[...].astype(jnp.bfloat16),
                preferred_element_type=jnp.float32) + b1_ref[...]
    a = z * jax.nn.sigmoid(z)
    o_ref[...] = jnp.dot(a.astype(jnp.bfloat16), w2_ref[...].astype(jnp.bfloat16),
                         preferred_element_type=jnp.float32) + b2_ref[...]


def _phi(h_i, W1, b1, W2, b2):
    blk = 1000
    return pl.pallas_call(
        _phi_body,
        grid=(N // blk,),
        in_specs=[
            pl.BlockSpec((blk, FEAT), lambda i: (i, 0)),
            pl.BlockSpec((FEAT, FEAT), lambda i: (0, 0)),
            pl.BlockSpec((1, FEAT), lambda i: (0, 0)),
            pl.BlockSpec((FEAT, FEAT), lambda i: (0, 0)),
            pl.BlockSpec((1, FEAT), lambda i: (0, 0)),
        ],
        out_specs=pl.BlockSpec((blk, FEAT), lambda i: (i, 0)),
        out_shape=jax.ShapeDtypeStruct((N, FEAT), jnp.float32),
    )(h_i, W1, b1.reshape(1, FEAT), W2, b2.reshape(1, FEAT))


def _gather_body(phi_hbm, src_hbm, out_hbm, idx_v, rows_v, sem):
    wid = lax.axis_index("s") * NCORE + lax.axis_index("c")
    nround = (NCHUNKG + NW - 1) // NW

    def round_body(r, carry):
        chunk = r * NW + wid

        @pl.when(chunk < NCHUNKG)
        def _():
            e0 = chunk * CHG
            pltpu.sync_copy(src_hbm.at[pl.ds(e0, CHG)], idx_v)
            pltpu.async_copy(phi_hbm.at[idx_v], rows_v, sem).wait()
            pltpu.sync_copy(rows_v, out_hbm.at[pl.ds(e0, CHG)])
        return carry

    lax.fori_loop(0, nround, round_body, 0)


def _gather_phi(phi, src):
    mesh = plsc.VectorSubcoreMesh(core_axis_name="c", subcore_axis_name="s")
    k = pl.kernel(
        _gather_body,
        out_type=jax.ShapeDtypeStruct((E, FEAT), jnp.float32),
        mesh=mesh,
        scratch_types=[
            pltpu.VMEM((CHG,), jnp.int32),
            pltpu.VMEM((CHG, FEAT), jnp.float32),
            pltpu.SemaphoreType.DMA,
        ],
    )
    return k(phi, src)


def _edge_body(d_ref, g_ref, we1_ref, be1_ref, we2_ref, be2_ref,
               wf0_ref, bf0_ref, wf1_ref, bf1_ref, f0_ref, f1_ref):
    d = d_ref[...]
    step = CUTOFF / (NRBF - 1)
    offs = lax.broadcasted_iota(jnp.int32, (1, NRBF), 1).astype(jnp.float32) * step
    coeff = -0.5 / (step * step)
    smear = jnp.exp(coeff * jnp.square(d - offs))
    h = _softplus(jnp.dot(smear.astype(jnp.bfloat16),
                          we1_ref[...].astype(jnp.bfloat16),
                          preferred_element_type=jnp.float32)
                  + be1_ref[...]) - 0.6931471805599453
    emb = jnp.dot(h.astype(jnp.bfloat16), we2_ref[...].astype(jnp.bfloat16),
                  preferred_element_type=jnp.float32) + be2_ref[...]
    ei = (g_ref[...] * emb).astype(jnp.bfloat16)
    f0_ref[...] = jnp.dot(ei, wf0_ref[...].astype(jnp.bfloat16),
                          preferred_element_type=jnp.float32) + bf0_ref[...]
    f1_ref[...] = jnp.dot(ei, wf1_ref[...].astype(jnp.bfloat16),
                          preferred_element_type=jnp.float32) + bf1_ref[...]


def _edge_filters(d_ij, G, We1, be1, We2, be2, Wf0, bf0, Wf1, bf1):
    blk = 512
    grid = (E + blk - 1) // blk
    full = lambda shape: pl.BlockSpec(shape, lambda i: (0, 0))
    return pl.pallas_call(
        _edge_body,
        grid=(grid,),
        in_specs=[
            pl.BlockSpec((blk, 1), lambda i: (i, 0)),
            pl.BlockSpec((blk, FEAT), lambda i: (i, 0)),
            full((NRBF, FEAT)), full((1, FEAT)),
            full((FEAT, FEAT)), full((1, FEAT)),
            full((FEAT, FEAT)), full((1, FEAT)),
            full((FEAT, FEAT)), full((1, FEAT)),
        ],
        out_specs=[pl.BlockSpec((blk, FEAT), lambda i: (i, 0)),
                   pl.BlockSpec((blk, FEAT), lambda i: (i, 0))],
        out_shape=[jax.ShapeDtypeStruct((E, FEAT), jnp.float32),
                   jax.ShapeDtypeStruct((E, FEAT), jnp.float32)],
    )(d_ij.reshape(E, 1), G, We1, be1.reshape(1, FEAT), We2, be2.reshape(1, FEAT),
      Wf0, bf0.reshape(1, FEAT), Wf1, bf1.reshape(1, FEAT))


def _splat(vec, i):
    dnums = lax.GatherDimensionNumbers(
        offset_dims=(), collapsed_slice_dims=(0,), start_index_map=(0,))
    idx = jnp.full((16, 1), i, jnp.int32)
    return lax.gather(vec, idx, dnums, (1,),
                      mode=lax.GatherScatterMode.PROMISE_IN_BOUNDS)


_A_KEYS = ("src", "dst", "s1i", "f0", "f1", "u", "vg", "dv", "seml")


def _vscat_a_body(vT, f0h, f1h, uflat, src, dst, vout, acc, *bufs):
    c = lax.axis_index("c")
    s = lax.axis_index("s")

    sets = []
    for b in range(2):
        d = dict(zip(_A_KEYS, bufs[b * 9:b * 9 + 9]))
        d["semg"] = bufs[18 + 2 * b]
        d["sems"] = bufs[19 + 2 * b]
        sets.append(d)

    n0 = s * ROWS_A
    pltpu.sync_copy(vT.at[pl.ds(c * N + n0, ROWS_A)], acc.at[pl.ds(n0, ROWS_A)])

    @pl.when(s == NSUB - 1)
    def _():
        pltpu.sync_copy(vT.at[pl.ds(c * N + NSUB * ROWS_A, TAIL_A)],
                        acc.at[pl.ds(NSUB * ROWS_A, TAIL_A)])

    plsc.subcore_barrier()
    nround = (NCHUNK + NSUB - 1) // NSUB

    def issue_l(jj, S):
        ch = jj * NSUB + s

        @pl.when(ch < NCHUNK)
        def _():
            e0 = ch * CH
            pltpu.async_copy(src.at[pl.ds(e0, CH)], S["src"], S["seml"])
            pltpu.async_copy(dst.at[pl.ds(e0, CH)], S["dst"], S["seml"])
            pltpu.async_copy(f0h.at[pl.ds(e0, CH)], S["f0"], S["seml"])
            pltpu.async_copy(f1h.at[pl.ds(e0, CH)], S["f1"], S["seml"])
            pltpu.async_copy(uflat.at[pl.ds(e0 * 16, CH * 16)], S["u"], S["seml"])

    def issue_g(jj, S):
        ch = jj * NSUB + s

        @pl.when(ch < NCHUNK)
        def _():
            pltpu.make_async_copy(src.at[pl.ds(0, CH)], S["src"], S["seml"]).wait()
            pltpu.make_async_copy(dst.at[pl.ds(0, CH)], S["dst"], S["seml"]).wait()
            pltpu.make_async_copy(f0h.at[pl.ds(0, CH)], S["f0"], S["seml"]).wait()
            pltpu.make_async_copy(f1h.at[pl.ds(0, CH)], S["f1"], S["seml"]).wait()
            pltpu.make_async_copy(uflat.at[pl.ds(0, CH * 16)], S["u"], S["seml"]).wait()
            for i in range(CH // 16):
                sl = pl.ds(i * 16, 16)
                S["dst"][sl] = S["dst"][sl] + c * N
            pltpu.async_copy(vT.at[S["dst"]], S["vg"], S["semg"])

    def do_c(jj, S):
        ch = jj * NSUB + s

        @pl.when(ch < NCHUNK)
        def _():
            @pl.when(jj >= 2)
            def _():
                pltpu.make_async_copy(S["dv"], acc.at[S["s1i"]], S["sems"]).wait()
            pltpu.make_async_copy(vT.at[S["dst"]], S["vg"], S["semg"]).wait()
            for i in range(CH // 16):
                sl = pl.ds(i * 16, 16)
                S["s1i"][sl] = S["src"][sl]

            @plsc.parallel_loop(0, CH, 1, unroll=4)
            def edge_body(e):
                uv = S["u"][pl.ds(e * 16, 16)]
                u_own = _splat(uv, c)
                for kg in range(FEAT // 16):
                    ksl = pl.ds(kg * 16, 16)
                    S["dv"][e, ksl] = (S["f0"][e, ksl] * u_own
                                       + S["f1"][e, ksl] * S["vg"][e, ksl])

            pltpu.async_copy(S["dv"], acc.at[S["s1i"]], S["sems"], add=True)

    issue_l(0, sets[0])
    issue_l(1, sets[1])
    issue_g(0, sets[0])

    def pair_body(j2, carry):
        for b in range(2):
            jj = 2 * j2 + b
            issue_g(jj + 1, sets[1 - b])
            do_c(jj, sets[b])
            issue_l(jj + 2, sets[b])
        return carry

    npairs = (nround + 1) // 2
    lax.fori_loop(0, npairs, pair_body, 0)
    for b in range(2):
        pltpu.make_async_copy(sets[b]["dv"], acc.at[sets[b]["s1i"]],
                              sets[b]["sems"]).wait()
    plsc.subcore_barrier()
    pltpu.sync_copy(acc.at[pl.ds(n0, ROWS_A)], vout.at[pl.ds(c * N + n0, ROWS_A)])

    @pl.when(s == NSUB - 1)
    def _():
        pltpu.sync_copy(acc.at[pl.ds(NSUB * ROWS_A, TAIL_A)],
                        vout.at[pl.ds(c * N + NSUB * ROWS_A, TAIL_A)])


def _vscatter_a(vT, f0, f1, uflat, src, dst):
    mesh = plsc.VectorSubcoreMesh(core_axis_name="c", subcore_axis_name="s")
    bufset = [
        pltpu.VMEM((CH,), jnp.int32),
        pltpu.VMEM((CH,), jnp.int32),
        pltpu.VMEM((CH,), jnp.int32),
        pltpu.VMEM((CH, FEAT), jnp.float32),
        pltpu.VMEM((CH, FEAT), jnp.float32),
        pltpu.VMEM((CH * 16,), jnp.float32),
        pltpu.VMEM((CH, FEAT), jnp.float32),
        pltpu.VMEM((CH, FEAT), jnp.float32),
        pltpu.SemaphoreType.DMA,
    ]
    k = pl.kernel(
        _vscat_a_body,
        out_type=jax.ShapeDtypeStruct((2 * N, FEAT), jnp.float32),
        mesh=mesh,
        scratch_types=[pltpu.VMEM_SHARED((N, FEAT), jnp.float32)]
        + bufset + bufset + [pltpu.SemaphoreType.DMA] * 4,
    )
    return k(vT, f0, f1, uflat, src, dst)


_B_KEYS = ("src", "dst", "s1i", "f0", "f1", "u", "vg", "dv", "hs", "seml")


def _vscat_b_body(vT, h_i, f0h, f1h, uflat, src, dst, v2out, hout, acc, *bufs):
    c = lax.axis_index("c")
    s = lax.axis_index("s")

    sets = []
    for b in range(2):
        d = dict(zip(_B_KEYS, bufs[b * 10:b * 10 + 10]))
        d["semg"] = bufs[20 + 2 * b]
        d["sems"] = bufs[21 + 2 * b]
        sets.append(d)

    n0 = s * ROWS_A

    @pl.when(c == 0)
    def _():
        pltpu.sync_copy(vT.at[pl.ds(2 * N + n0, ROWS_A)],
                        acc.at[pl.ds(n0, ROWS_A)])

        @pl.when(s == NSUB - 1)
        def _():
            pltpu.sync_copy(vT.at[pl.ds(2 * N + NSUB * ROWS_A, TAIL_A)],
                            acc.at[pl.ds(NSUB * ROWS_A, TAIL_A)])

    @pl.when(c == 1)
    def _():
        pltpu.sync_copy(h_i.at[pl.ds(n0, ROWS_A)], acc.at[pl.ds(n0, ROWS_A)])

        @pl.when(s == NSUB - 1)
        def _():
            pltpu.sync_copy(h_i.at[pl.ds(NSUB * ROWS_A, TAIL_A)],
                            acc.at[pl.ds(NSUB * ROWS_A, TAIL_A)])

    plsc.subcore_barrier()
    nround = (NCHUNK + NSUB - 1) // NSUB

    def issue_l(jj, S):
        ch = jj * NSUB + s

        @pl.when(ch < NCHUNK)
        def _():
            e0 = ch * CH
            pltpu.async_copy(src.at[pl.ds(e0, CH)], S["src"], S["seml"])
            pltpu.async_copy(f1h.at[pl.ds(e0, CH)], S["f1"], S["seml"])

            @pl.when(c == 0)
            def _():
                pltpu.async_copy(dst.at[pl.ds(e0, CH)], S["dst"], S["seml"])
                pltpu.async_copy(f0h.at[pl.ds(e0, CH)], S["f0"], S["seml"])
                pltpu.async_copy(uflat.at[pl.ds(e0 * 16, CH * 16)], S["u"],
                                 S["seml"])

    def issue_g(jj, S):
        ch = jj * NSUB + s

        @pl.when(ch < NCHUNK)
        def _():
            pltpu.make_async_copy(src.at[pl.ds(0, CH)], S["src"], S["seml"]).wait()
            pltpu.make_async_copy(f1h.at[pl.ds(0, CH)], S["f1"], S["seml"]).wait()

            @pl.when(c == 0)
            def _():
                pltpu.make_async_copy(dst.at[pl.ds(0, CH)], S["dst"],
                                      S["seml"]).wait()
                pltpu.make_async_copy(f0h.at[pl.ds(0, CH)], S["f0"],
                                      S["seml"]).wait()
                pltpu.make_async_copy(uflat.at[pl.ds(0, CH * 16)], S["u"],
                                      S["seml"]).wait()
                for i in range(CH // 16):
                    sl = pl.ds(i * 16, 16)
                    S["dst"][sl] = S["dst"][sl] + 2 * N
                pltpu.async_copy(vT.at[S["dst"]], S["vg"], S["semg"])

    def do_c(jj, S):
        ch = jj * NSUB + s

        @pl.when(ch < NCHUNK)
        def _():
            @pl.when(jj >= 2)
            def _():
                pltpu.make_async_copy(S["dv"], acc.at[S["s1i"]], S["sems"]).wait()
            for i in range(CH // 16):
                sl = pl.ds(i * 16, 16)
                S["s1i"][sl] = S["src"][sl]

            @pl.when(c == 0)
            def _():
                pltpu.make_async_copy(vT.at[S["dst"]], S["vg"], S["semg"]).wait()

                @plsc.parallel_loop(0, CH, 1, unroll=4)
                def edge_body(e):
                    uv = S["u"][pl.ds(e * 16, 16)]
                    u_2 = _splat(uv, 2)
                    for kg in range(FEAT // 16):
                        ksl = pl.ds(kg * 16, 16)
                        S["dv"][e, ksl] = (S["f0"][e, ksl] * u_2
                                           + S["f1"][e, ksl] * S["vg"][e, ksl])

                pltpu.async_copy(S["dv"], acc.at[S["s1i"]], S["sems"], add=True)

            @pl.when(c == 1)
            def _():
                @plsc.parallel_loop(0, CH, 1, unroll=4)
                def edge_body(e):
                    for kg in range(FEAT // 16):
                        ksl = pl.ds(kg * 16, 16)
                        S["hs"][e, ksl] = S["f1"][e, ksl]

                pltpu.async_copy(S["hs"], acc.at[S["s1i"]], S["sems"], add=True)

    issue_l(0, sets[0])
    issue_l(1, sets[1])
    issue_g(0, sets[0])

    def pair_body(j2, carry):
        for b in range(2):
            jj = 2 * j2 + b
            issue_g(jj + 1, sets[1 - b])
            do_c(jj, sets[b])
            issue_l(jj + 2, sets[b])
        return carry

    npairs = (nround + 1) // 2
    lax.fori_loop(0, npairs, pair_body, 0)
    for b in range(2):
        pltpu.make_async_copy(sets[b]["dv"], acc.at[sets[b]["s1i"]],
                              sets[b]["sems"]).wait()
    plsc.subcore_barrier()

    @pl.when(c == 0)
    def _():
        pltpu.sync_copy(acc.at[pl.ds(n0, ROWS_A)], v2out.at[pl.ds(n0, ROWS_A)])

        @pl.when(s == NSUB - 1)
        def _():
            pltpu.sync_copy(acc.at[pl.ds(NSUB * ROWS_A, TAIL_A)],
                            v2out.at[pl.ds(NSUB * ROWS_A, TAIL_A)])

    @pl.when(c == 1)
    def _():
        pltpu.sync_copy(acc.at[pl.ds(n0, ROWS_A)], hout.at[pl.ds(n0, ROWS_A)])

        @pl.when(s == NSUB - 1)
        def _():
            pltpu.sync_copy(acc.at[pl.ds(NSUB * ROWS_A, TAIL_A)],
                            hout.at[pl.ds(NSUB * ROWS_A, TAIL_A)])


def _vscatter_b(vT, h_i, f0, f1, uflat, src, dst):
    mesh = plsc.VectorSubcoreMesh(core_axis_name="c", subcore_axis_name="s")
    bufset = [
        pltpu.VMEM((CH,), jnp.int32),
        pltpu.VMEM((CH,), jnp.int32),
        pltpu.VMEM((CH,), jnp.int32),
        pltpu.VMEM((CH, FEAT), jnp.float32),
        pltpu.VMEM((CH, FEAT), jnp.float32),
        pltpu.VMEM((CH * 16,), jnp.float32),
        pltpu.VMEM((CH, FEAT), jnp.float32),
        pltpu.VMEM((CH, FEAT), jnp.float32),
        pltpu.VMEM((CH, FEAT), jnp.float32),
        pltpu.SemaphoreType.DMA,
    ]
    k = pl.kernel(
        _vscat_b_body,
        out_type=[jax.ShapeDtypeStruct((N, FEAT), jnp.float32),
                  jax.ShapeDtypeStruct((N, FEAT), jnp.float32)],
        mesh=mesh,
        scratch_types=[pltpu.VMEM_SHARED((N, FEAT), jnp.float32)]
        + bufset + bufset + [pltpu.SemaphoreType.DMA] * 4,
    )
    return k(vT, h_i, f0, f1, uflat, src, dst)


def kernel(h_i, v_i, d_ij, unit_r_ij, nbrs, W1, b1, W2, b2,
           Wf0, bf0, Wf1, bf1, Wf2, bf2, We1, be1, We2, be2):
    src = nbrs[:, 0]
    dst = nbrs[:, 1]

    phi = _phi(h_i, W1, b1, W2, b2)
    G = _gather_phi(phi, src)
    f0, f1 = _edge_filters(d_ij, G, We1, be1, We2, be2, Wf0, bf0, Wf1, bf1)

    vT = v_i.transpose(2, 0, 1).reshape(3 * N, FEAT)
    uflat = jnp.pad(unit_r_ij, ((0, 0), (0, 13))).reshape(16 * E)

    vout01 = _vscatter_a(vT, f0, f1, uflat, src, dst)
    vout2, h_out = _vscatter_b(vT, h_i, f0, f1, uflat, src, dst)

    v_out = jnp.stack([vout01[:N], vout01[N:], vout2], axis=0).transpose(1, 2, 0)
    return (h_out, v_out)


# speedup vs baseline: 1.0170x; 1.0170x over previous
import jax
import jax.numpy as jnp
from jax import lax
from jax.experimental import pallas as pl
from jax.experimental.pallas import tpu as pltpu
from jax.experimental.pallas import tpu_sc as plsc

N = 10000
E = 160000
FEAT = 128
NRBF = 50
CUTOFF = 5.0

CH = 32
NCHUNK = E // CH
CHG = 128
NCHUNKG = E // CHG
NSUB = 16
NCORE = 2
NW = NSUB * NCORE
NHALF = N // 2

HB = NHALF + 8
NACC2 = 2 * NHALF + 16

ROWS_A = 624
TAIL_A = N - NSUB * ROWS_A
ROWS_B = 312
TAIL_B = NHALF - NSUB * ROWS_B


def _softplus(x):
    return jnp.maximum(x, 0.0) + jnp.log1p(jnp.exp(-jnp.abs(x)))


def _phi_body(h_ref, w1_ref, b1_ref, w2_ref, b2_ref, o_ref):
    h = h_ref[...].astype(jnp.bfloat16)
    z = jnp.dot(h, w1_ref[...].astype(jnp.bfloat16),
                preferred_element_type=jnp.float32) + b1_ref[...]
    a = z * jax.nn.sigmoid(z)
    o_ref[...] = jnp.dot(a.astype(jnp.bfloat16), w2_ref[...].astype(jnp.bfloat16),
                         preferred_element_type=jnp.float32) + b2_ref[...]


def _phi(h_i, W1, b1, W2, b2):
    blk = 1000
    return pl.pallas_call(
        _phi_body,
        grid=(N // blk,),
        in_specs=[
            pl.BlockSpec((blk, FEAT), lambda i: (i, 0)),
            pl.BlockSpec((FEAT, FEAT), lambda i: (0, 0)),
            pl.BlockSpec((1, FEAT), lambda i: (0, 0)),
            pl.BlockSpec((FEAT, FEAT), lambda i: (0, 0)),
            pl.BlockSpec((1, FEAT), lambda i: (0, 0)),
        ],
        out_specs=pl.BlockSpec((blk, FEAT), lambda i: (i, 0)),
        out_shape=jax.ShapeDtypeStruct((N, FEAT), jnp.float32),
    )(h_i, W1, b1.reshape(1, FEAT), W2, b2.reshape(1, FEAT))


def _gather_body(phi_hbm, src_hbm, out_hbm, idx_v, rows_v, sem):
    wid = lax.axis_index("s") * NCORE + lax.axis_index("c")
    nround = (NCHUNKG + NW - 1) // NW

    def round_body(r, carry):
        chunk = r * NW + wid

        @pl.when(chunk < NCHUNKG)
        def _():
            e0 = chunk * CHG
            pltpu.sync_copy(src_hbm.at[pl.ds(e0, CHG)], idx_v)
            pltpu.async_copy(phi_hbm.at[idx_v], rows_v, sem).wait()
            pltpu.sync_copy(rows_v, out_hbm.at[pl.ds(e0, CHG)])
        return carry

    lax.fori_loop(0, nround, round_body, 0)


def _gather_phi(phi, src):
    mesh = plsc.VectorSubcoreMesh(core_axis_name="c", subcore_axis_name="s")
    k = pl.kernel(
        _gather_body,
        out_type=jax.ShapeDtypeStruct((E, FEAT), jnp.float32),
        mesh=mesh,
        scratch_types=[
            pltpu.VMEM((CHG,), jnp.int32),
            pltpu.VMEM((CHG, FEAT), jnp.float32),
            pltpu.SemaphoreType.DMA,
        ],
    )
    return k(phi, src)


def _edge_body(d_ref, g_ref, we1_ref, be1_ref, we2_ref, be2_ref,
               wf0_ref, bf0_ref, wf1_ref, bf1_ref, f0_ref, f1_ref):
    d = d_ref[...]
    step = CUTOFF / (NRBF - 1)
    offs = lax.broadcasted_iota(jnp.int32, (1, NRBF), 1).astype(jnp.float32) * step
    coeff = -0.5 / (step * step)
    smear = jnp.exp(coeff * jnp.square(d - offs))
    h = _softplus(jnp.dot(smear.astype(jnp.bfloat16),
                          we1_ref[...].astype(jnp.bfloat16),
                          preferred_element_type=jnp.float32)
                  + be1_ref[...]) - 0.6931471805599453
    emb = jnp.dot(h.astype(jnp.bfloat16), we2_ref[...].astype(jnp.bfloat16),
                  preferred_element_type=jnp.float32) + be2_ref[...]
    ei = (g_ref[...] * emb).astype(jnp.bfloat16)
    f0_ref[...] = jnp.dot(ei, wf0_ref[...].astype(jnp.bfloat16),
                          preferred_element_type=jnp.float32) + bf0_ref[...]
    f1_ref[...] = jnp.dot(ei, wf1_ref[...].astype(jnp.bfloat16),
                          preferred_element_type=jnp.float32) + bf1_ref[...]


def _edge_filters(d_ij, G, We1, be1, We2, be2, Wf0, bf0, Wf1, bf1):
    blk = 512
    grid = (E + blk - 1) // blk
    full = lambda shape: pl.BlockSpec(shape, lambda i: (0, 0))
    return pl.pallas_call(
        _edge_body,
        grid=(grid,),
        in_specs=[
            pl.BlockSpec((blk, 1), lambda i: (i, 0)),
            pl.BlockSpec((blk, FEAT), lambda i: (i, 0)),
            full((NRBF, FEAT)), full((1, FEAT)),
            full((FEAT, FEAT)), full((1, FEAT)),
            full((FEAT, FEAT)), full((1, FEAT)),
            full((FEAT, FEAT)), full((1, FEAT)),
        ],
        out_specs=[pl.BlockSpec((blk, FEAT), lambda i: (i, 0)),
                   pl.BlockSpec((blk, FEAT), lambda i: (i, 0))],
        out_shape=[jax.ShapeDtypeStruct((E, FEAT), jnp.float32),
                   jax.ShapeDtypeStruct((E, FEAT), jnp.float32)],
    )(d_ij.reshape(E, 1), G, We1, be1.reshape(1, FEAT), We2, be2.reshape(1, FEAT),
      Wf0, bf0.reshape(1, FEAT), Wf1, bf1.reshape(1, FEAT))


def _splat(vec, i):
    dnums = lax.GatherDimensionNumbers(
        offset_dims=(), collapsed_slice_dims=(0,), start_index_map=(0,))
    idx = jnp.full((16, 1), i, jnp.int32)
    return lax.gather(vec, idx, dnums, (1,),
                      mode=lax.GatherScatterMode.PROMISE_IN_BOUNDS)


_A_KEYS = ("src", "dst", "s1i", "f0", "f1", "u", "vg", "dv", "seml")


def _vscat_a_body(vT, f0h, f1h, uflat, src, dst, vout, acc, *bufs):
    c = lax.axis_index("c")
    s = lax.axis_index("s")

    sets = []
    for b in range(2):
        d = dict(zip(_A_KEYS, bufs[b * 9:b * 9 + 9]))
        d["semg"] = bufs[18 + 2 * b]
        d["sems"] = bufs[19 + 2 * b]
        sets.append(d)

    n0 = s * ROWS_A
    pltpu.sync_copy(vT.at[pl.ds(c * N + n0, ROWS_A)], acc.at[pl.ds(n0, ROWS_A)])

    @pl.when(s == NSUB - 1)
    def _():
        pltpu.sync_copy(vT.at[pl.ds(c * N + NSUB * ROWS_A, TAIL_A)],
                        acc.at[pl.ds(NSUB * ROWS_A, TAIL_A)])

    plsc.subcore_barrier()
    nround = (NCHUNK + NSUB - 1) // NSUB

    def issue_l(jj, S):
        ch = jj * NSUB + s

        @pl.when(ch < NCHUNK)
        def _():
            e0 = ch * CH
            pltpu.async_copy(src.at[pl.ds(e0, CH)], S["src"], S["seml"])
            pltpu.async_copy(dst.at[pl.ds(e0, CH)], S["dst"], S["seml"])
            pltpu.async_copy(f0h.at[pl.ds(e0, CH)], S["f0"], S["seml"])
            pltpu.async_copy(f1h.at[pl.ds(e0, CH)], S["f1"], S["seml"])
            pltpu.async_copy(uflat.at[pl.ds(e0 * 16, CH * 16)], S["u"], S["seml"])

    def issue_g(jj, S):
        ch = jj * NSUB + s

        @pl.when(ch < NCHUNK)
        def _():
            pltpu.make_async_copy(src.at[pl.ds(0, CH)], S["src"], S["seml"]).wait()
            pltpu.make_async_copy(dst.at[pl.ds(0, CH)], S["dst"], S["seml"]).wait()
            pltpu.make_async_copy(f0h.at[pl.ds(0, CH)], S["f0"], S["seml"]).wait()
            pltpu.make_async_copy(f1h.at[pl.ds(0, CH)], S["f1"], S["seml"]).wait()
            pltpu.make_async_copy(uflat.at[pl.ds(0, CH * 16)], S["u"], S["seml"]).wait()
            for i in range(CH // 16):
                sl = pl.ds(i * 16, 16)
                S["dst"][sl] = S["dst"][sl] + c * N
            pltpu.async_copy(vT.at[S["dst"]], S["vg"], S["semg"])

    def do_c(jj, S):
        ch = jj * NSUB + s

        @pl.when(ch < NCHUNK)
        def _():
            @pl.when(jj >= 2)
            def _():
                pltpu.make_async_copy(S["dv"], acc.at[S["s1i"]], S["sems"]).wait()
            pltpu.make_async_copy(vT.at[S["dst"]], S["vg"], S["semg"]).wait()
            for i in range(CH // 16):
                sl = pl.ds(i * 16, 16)
                S["s1i"][sl] = S["src"][sl]

            @plsc.parallel_loop(0, CH, 1, unroll=4)
            def edge_body(e):
                uv = S["u"][pl.ds(e * 16, 16)]
                u_own = _splat(uv, c)
                for kg in range(FEAT // 16):
                    ksl = pl.ds(kg * 16, 16)
                    S["dv"][e, ksl] = (S["f0"][e, ksl] * u_own
                                       + S["f1"][e, ksl] * S["vg"][e, ksl])

            pltpu.async_copy(S["dv"], acc.at[S["s1i"]], S["sems"], add=True)

    issue_l(0, sets[0])
    issue_l(1, sets[1])
    issue_g(0, sets[0])

    def pair_body(j2, carry):
        for b in range(2):
            jj = 2 * j2 + b
            do_c(jj, sets[b])
            issue_l(jj + 2, sets[b])
            issue_g(jj + 1, sets[1 - b])
        return carry

    npairs = (nround + 1) // 2
    lax.fori_loop(0, npairs, pair_body, 0)
    for b in range(2):
        pltpu.make_async_copy(sets[b]["dv"], acc.at[sets[b]["s1i"]],
                              sets[b]["sems"]).wait()
    plsc.subcore_barrier()
    pltpu.sync_copy(acc.at[pl.ds(n0, ROWS_A)], vout.at[pl.ds(c * N + n0, ROWS_A)])

    @pl.when(s == NSUB - 1)
    def _():
        pltpu.sync_copy(acc.at[pl.ds(NSUB * ROWS_A, TAIL_A)],
                        vout.at[pl.ds(c * N + NSUB * ROWS_A, TAIL_A)])


def _vscatter_a(vT, f0, f1, uflat, src, dst):
    mesh = plsc.VectorSubcoreMesh(core_axis_name="c", subcore_axis_name="s")
    bufset = [
        pltpu.VMEM((CH,), jnp.int32),
        pltpu.VMEM((CH,), jnp.int32),
        pltpu.VMEM((CH,), jnp.int32),
        pltpu.VMEM((CH, FEAT), jnp.float32),
        pltpu.VMEM((CH, FEAT), jnp.float32),
        pltpu.VMEM((CH * 16,), jnp.float32),
        pltpu.VMEM((CH, FEAT), jnp.float32),
        pltpu.VMEM((CH, FEAT), jnp.float32),
        pltpu.SemaphoreType.DMA,
    ]
    k = pl.kernel(
        _vscat_a_body,
        out_type=jax.ShapeDtypeStruct((2 * N, FEAT), jnp.float32),
        mesh=mesh,
        scratch_types=[pltpu.VMEM_SHARED((N, FEAT), jnp.float32)]
        + bufset + bufset + [pltpu.SemaphoreType.DMA] * 4,
    )
    return k(vT, f0, f1, uflat, src, dst)


_B_KEYS = ("src", "dst", "s1i", "f0", "f1", "u", "vg", "dv", "hs", "seml")


def _vscat_b_body(vT, h_i, f0h, f1h, uflat, src, dst, v2out, hout, acc, *bufs):
    c = lax.axis_index("c")
    s = lax.axis_index("s")

    sets = []
    for b in range(2):
        d = dict(zip(_B_KEYS, bufs[b * 10:b * 10 + 10]))
        d["semg"] = bufs[20 + 2 * b]
        d["sems"] = bufs[21 + 2 * b]
        sets.append(d)

    n0 = s * ROWS_A

    @pl.when(c == 0)
    def _():
        pltpu.sync_copy(vT.at[pl.ds(2 * N + n0, ROWS_A)],
                        acc.at[pl.ds(n0, ROWS_A)])

        @pl.when(s == NSUB - 1)
        def _():
            pltpu.sync_copy(vT.at[pl.ds(2 * N + NSUB * ROWS_A, TAIL_A)],
                            acc.at[pl.ds(NSUB * ROWS_A, TAIL_A)])

    @pl.when(c == 1)
    def _():
        pltpu.sync_copy(h_i.at[pl.ds(n0, ROWS_A)], acc.at[pl.ds(n0, ROWS_A)])

        @pl.when(s == NSUB - 1)
        def _():
            pltpu.sync_copy(h_i.at[pl.ds(NSUB * ROWS_A, TAIL_A)],
                            acc.at[pl.ds(NSUB * ROWS_A, TAIL_A)])

    plsc.subcore_barrier()
    nround = (NCHUNK + NSUB - 1) // NSUB

    def issue_l(jj, S):
        ch = jj * NSUB + s

        @pl.when(ch < NCHUNK)
        def _():
            e0 = ch * CH
            pltpu.async_copy(src.at[pl.ds(e0, CH)], S["src"], S["seml"])
            pltpu.async_copy(f1h.at[pl.ds(e0, CH)], S["f1"], S["seml"])

            @pl.when(c == 0)
            def _():
                pltpu.async_copy(dst.at[pl.ds(e0, CH)], S["dst"], S["seml"])
                pltpu.async_copy(f0h.at[pl.ds(e0, CH)], S["f0"], S["seml"])
                pltpu.async_copy(uflat.at[pl.ds(e0 * 16, CH * 16)], S["u"],
                                 S["seml"])

    def issue_g(jj, S):
        ch = jj * NSUB + s

        @pl.when(ch < NCHUNK)
        def _():
            pltpu.make_async_copy(src.at[pl.ds(0, CH)], S["src"], S["seml"]).wait()
            pltpu.make_async_copy(f1h.at[pl.ds(0, CH)], S["f1"], S["seml"]).wait()

            @pl.when(c == 0)
            def _():
                pltpu.make_async_copy(dst.at[pl.ds(0, CH)], S["dst"],
                                      S["seml"]).wait()
                pltpu.make_async_copy(f0h.at[pl.ds(0, CH)], S["f0"],
                                      S["seml"]).wait()
                pltpu.make_async_copy(uflat.at[pl.ds(0, CH * 16)], S["u"],
                                      S["seml"]).wait()
                for i in range(CH // 16):
                    sl = pl.ds(i * 16, 16)
                    S["dst"][sl] = S["dst"][sl] + 2 * N
                pltpu.async_copy(vT.at[S["dst"]], S["vg"], S["semg"])

    def do_c(jj, S):
        ch = jj * NSUB + s

        @pl.when(ch < NCHUNK)
        def _():
            @pl.when(jj >= 2)
            def _():
                pltpu.make_async_copy(S["dv"], acc.at[S["s1i"]], S["sems"]).wait()
            for i in range(CH // 16):
                sl = pl.ds(i * 16, 16)
                S["s1i"][sl] = S["src"][sl]

            @pl.when(c == 0)
            def _():
                pltpu.make_async_copy(vT.at[S["dst"]], S["vg"], S["semg"]).wait()

                @plsc.parallel_loop(0, CH, 1, unroll=4)
                def edge_body(e):
                    uv = S["u"][pl.ds(e * 16, 16)]
                    u_2 = _splat(uv, 2)
                    for kg in range(FEAT // 16):
                        ksl = pl.ds(kg * 16, 16)
                        S["dv"][e, ksl] = (S["f0"][e, ksl] * u_2
                                           + S["f1"][e, ksl] * S["vg"][e, ksl])

                pltpu.async_copy(S["dv"], acc.at[S["s1i"]], S["sems"], add=True)

            @pl.when(c == 1)
            def _():
                @plsc.parallel_loop(0, CH, 1, unroll=4)
                def edge_body(e):
                    for kg in range(FEAT // 16):
                        ksl = pl.ds(kg * 16, 16)
                        S["hs"][e, ksl] = S["f1"][e, ksl]

                pltpu.async_copy(S["hs"], acc.at[S["s1i"]], S["sems"], add=True)

    issue_l(0, sets[0])
    issue_l(1, sets[1])
    issue_g(0, sets[0])

    def pair_body(j2, carry):
        for b in range(2):
            jj = 2 * j2 + b
            do_c(jj, sets[b])
            issue_l(jj + 2, sets[b])
            issue_g(jj + 1, sets[1 - b])
        return carry

    npairs = (nround + 1) // 2
    lax.fori_loop(0, npairs, pair_body, 0)
    for b in range(2):
        pltpu.make_async_copy(sets[b]["dv"], acc.at[sets[b]["s1i"]],
                              sets[b]["sems"]).wait()
    plsc.subcore_barrier()

    @pl.when(c == 0)
    def _():
        pltpu.sync_copy(acc.at[pl.ds(n0, ROWS_A)], v2out.at[pl.ds(n0, ROWS_A)])

        @pl.when(s == NSUB - 1)
        def _():
            pltpu.sync_copy(acc.at[pl.ds(NSUB * ROWS_A, TAIL_A)],
                            v2out.at[pl.ds(NSUB * ROWS_A, TAIL_A)])

    @pl.when(c == 1)
    def _():
        pltpu.sync_copy(acc.at[pl.ds(n0, ROWS_A)], hout.at[pl.ds(n0, ROWS_A)])

        @pl.when(s == NSUB - 1)
        def _():
            pltpu.sync_copy(acc.at[pl.ds(NSUB * ROWS_A, TAIL_A)],
                            hout.at[pl.ds(NSUB * ROWS_A, TAIL_A)])


def _vscatter_b(vT, h_i, f0, f1, uflat, src, dst):
    mesh = plsc.VectorSubcoreMesh(core_axis_name="c", subcore_axis_name="s")
    bufset = [
        pltpu.VMEM((CH,), jnp.int32),
        pltpu.VMEM((CH,), jnp.int32),
        pltpu.VMEM((CH,), jnp.int32),
        pltpu.VMEM((CH, FEAT), jnp.float32),
        pltpu.VMEM((CH, FEAT), jnp.float32),
        pltpu.VMEM((CH * 16,), jnp.float32),
        pltpu.VMEM((CH, FEAT), jnp.float32),
        pltpu.VMEM((CH, FEAT), jnp.float32),
        pltpu.VMEM((CH, FEAT), jnp.float32),
        pltpu.SemaphoreType.DMA,
    ]
    k = pl.kernel(
        _vscat_b_body,
        out_type=[jax.ShapeDtypeStruct((N, FEAT), jnp.float32),
                  jax.ShapeDtypeStruct((N, FEAT), jnp.float32)],
        mesh=mesh,
        scratch_types=[pltpu.VMEM_SHARED((N, FEAT), jnp.float32)]
        + bufset + bufset + [pltpu.SemaphoreType.DMA] * 4,
    )
    return k(vT, h_i, f0, f1, uflat, src, dst)


def kernel(h_i, v_i, d_ij, unit_r_ij, nbrs, W1, b1, W2, b2,
           Wf0, bf0, Wf1, bf1, Wf2, bf2, We1, be1, We2, be2):
    src = nbrs[:, 0]
    dst = nbrs[:, 1]

    phi = _phi(h_i, W1, b1, W2, b2)
    G = _gather_phi(phi, src)
    f0, f1 = _edge_filters(d_ij, G, We1, be1, We2, be2, Wf0, bf0, Wf1, bf1)

    vT = v_i.transpose(2, 0, 1).reshape(3 * N, FEAT)
    uflat = jnp.pad(unit_r_ij, ((0, 0), (0, 13))).reshape(16 * E)

    vout01 = _vscatter_a(vT, f0, f1, uflat, src, dst)
    vout2, h_out = _vscatter_b(vT, h_i, f0, f1, uflat, src, dst)

    v_out = jnp.stack([vout01[:N], vout01[N:], vout2], axis=0).transpose(1, 2, 0)
    return (h_out, v_out)


# speedup vs baseline: 1.0176x; 1.0005x over previous
import jax
import jax.numpy as jnp
from jax import lax
from jax.experimental import pallas as pl
from jax.experimental.pallas import tpu as pltpu
from jax.experimental.pallas import tpu_sc as plsc

N = 10000
E = 160000
FEAT = 128
NRBF = 50
CUTOFF = 5.0

CH = 32
NCHUNK = E // CH
CHG = 128
NCHUNKG = E // CHG
NSUB = 16
NCORE = 2
NW = NSUB * NCORE
ROWS_A = 624
TAIL_A = N - NSUB * ROWS_A


def _softplus(x):
    return jnp.maximum(x, 0.0) + jnp.log1p(jnp.exp(-jnp.abs(x)))


def _phi_body(h_ref, w1_ref, b1_ref, w2_ref, b2_ref, o_ref):
    h = h_ref[...].astype(jnp.bfloat16)
    z = jnp.dot(h, w1_ref[...].astype(jnp.bfloat16),
                preferred_element_type=jnp.float32) + b1_ref[...]
    a = z * jax.nn.sigmoid(z)
    o_ref[...] = jnp.dot(a.astype(jnp.bfloat16), w2_ref[...].astype(jnp.bfloat16),
                         preferred_element_type=jnp.float32) + b2_ref[...]


def _phi(h_i, W1, b1, W2, b2):
    blk = 1000
    return pl.pallas_call(
        _phi_body,
        grid=(N // blk,),
        in_specs=[
            pl.BlockSpec((blk, FEAT), lambda i: (i, 0)),
            pl.BlockSpec((FEAT, FEAT), lambda i: (0, 0)),
            pl.BlockSpec((1, FEAT), lambda i: (0, 0)),
            pl.BlockSpec((FEAT, FEAT), lambda i: (0, 0)),
            pl.BlockSpec((1, FEAT), lambda i: (0, 0)),
        ],
        out_specs=pl.BlockSpec((blk, FEAT), lambda i: (i, 0)),
        out_shape=jax.ShapeDtypeStruct((N, FEAT), jnp.float32),
    )(h_i, W1, b1.reshape(1, FEAT), W2, b2.reshape(1, FEAT))


def _gather_body(phi_hbm, src_hbm, out_hbm, idx_v, rows_v, sem):
    wid = lax.axis_index("s") * NCORE + lax.axis_index("c")
    nround = (NCHUNKG + NW - 1) // NW

    def round_body(r, carry):
        chunk = r * NW + wid

        @pl.when(chunk < NCHUNKG)
        def _():
            e0 = chunk * CHG
            pltpu.sync_copy(src_hbm.at[pl.ds(e0, CHG)], idx_v)
            pltpu.async_copy(phi_hbm.at[idx_v], rows_v, sem).wait()
            pltpu.sync_copy(rows_v, out_hbm.at[pl.ds(e0, CHG)])
        return carry

    lax.fori_loop(0, nround, round_body, 0)


def _gather_phi(phi, src):
    mesh = plsc.VectorSubcoreMesh(core_axis_name="c", subcore_axis_name="s")
    k = pl.kernel(
        _gather_body,
        out_type=jax.ShapeDtypeStruct((E, FEAT), jnp.float32),
        mesh=mesh,
        scratch_types=[
            pltpu.VMEM((CHG,), jnp.int32),
            pltpu.VMEM((CHG, FEAT), jnp.float32),
            pltpu.SemaphoreType.DMA,
        ],
    )
    return k(phi, src)


def _edge_body(d_ref, g_ref, we1_ref, be1_ref, we2_ref, be2_ref,
               wf0_ref, bf0_ref, wf1_ref, bf1_ref, f0_ref, f1_ref):
    d = d_ref[...]
    step = CUTOFF / (NRBF - 1)
    offs = lax.broadcasted_iota(jnp.int32, (1, NRBF), 1).astype(jnp.float32) * step
    coeff = -0.5 / (step * step)
    smear = jnp.exp(coeff * jnp.square(d - offs))
    h = _softplus(jnp.dot(smear.astype(jnp.bfloat16),
                          we1_ref[...].astype(jnp.bfloat16),
                          preferred_element_type=jnp.float32)
                  + be1_ref[...]) - 0.6931471805599453
    emb = jnp.dot(h.astype(jnp.bfloat16), we2_ref[...].astype(jnp.bfloat16),
                  preferred_element_type=jnp.float32) + be2_ref[...]
    ei = (g_ref[...] * emb).astype(jnp.bfloat16)
    f0_ref[...] = jnp.dot(ei, wf0_ref[...].astype(jnp.bfloat16),
                          preferred_element_type=jnp.float32) + bf0_ref[...]
    f1_ref[...] = jnp.dot(ei, wf1_ref[...].astype(jnp.bfloat16),
                          preferred_element_type=jnp.float32) + bf1_ref[...]


def _edge_filters(d_ij, G, We1, be1, We2, be2, Wf0, bf0, Wf1, bf1):
    blk = 512
    grid = (E + blk - 1) // blk
    full = lambda shape: pl.BlockSpec(shape, lambda i: (0, 0))
    return pl.pallas_call(
        _edge_body,
        grid=(grid,),
        in_specs=[
            pl.BlockSpec((blk, 1), lambda i: (i, 0)),
            pl.BlockSpec((blk, FEAT), lambda i: (i, 0)),
            full((NRBF, FEAT)), full((1, FEAT)),
            full((FEAT, FEAT)), full((1, FEAT)),
            full((FEAT, FEAT)), full((1, FEAT)),
            full((FEAT, FEAT)), full((1, FEAT)),
        ],
        out_specs=[pl.BlockSpec((blk, FEAT), lambda i: (i, 0)),
                   pl.BlockSpec((blk, FEAT), lambda i: (i, 0))],
        out_shape=[jax.ShapeDtypeStruct((E, FEAT), jnp.float32),
                   jax.ShapeDtypeStruct((E, FEAT), jnp.float32)],
    )(d_ij.reshape(E, 1), G, We1, be1.reshape(1, FEAT), We2, be2.reshape(1, FEAT),
      Wf0, bf0.reshape(1, FEAT), Wf1, bf1.reshape(1, FEAT))


def _splat(vec, i):
    dnums = lax.GatherDimensionNumbers(
        offset_dims=(), collapsed_slice_dims=(0,), start_index_map=(0,))
    idx = jnp.full((16, 1), i, jnp.int32)
    return lax.gather(vec, idx, dnums, (1,),
                      mode=lax.GatherScatterMode.PROMISE_IN_BOUNDS)


_A_KEYS = ("src", "dst", "s1i", "f0", "f1", "u", "vg", "dv", "seml")


def _vscat_a_body(vT, f0h, f1h, uflat, src, dst, vout, acc, *bufs):
    c = lax.axis_index("c")
    s = lax.axis_index("s")

    sets = []
    for b in range(2):
        d = dict(zip(_A_KEYS, bufs[b * 9:b * 9 + 9]))
        d["semg"] = bufs[18 + 2 * b]
        d["sems"] = bufs[19 + 2 * b]
        sets.append(d)

    n0 = s * ROWS_A
    pltpu.sync_copy(vT.at[pl.ds(c * N + n0, ROWS_A)], acc.at[pl.ds(n0, ROWS_A)])

    @pl.when(s == NSUB - 1)
    def _():
        pltpu.sync_copy(vT.at[pl.ds(c * N + NSUB * ROWS_A, TAIL_A)],
                        acc.at[pl.ds(NSUB * ROWS_A, TAIL_A)])

    plsc.subcore_barrier()
    nround = (NCHUNK + NSUB - 1) // NSUB

    def issue_l(jj, S):
        ch = jj * NSUB + s

        @pl.when(ch < NCHUNK)
        def _():
            e0 = ch * CH
            pltpu.async_copy(src.at[pl.ds(e0, CH)], S["src"], S["seml"])
            pltpu.async_copy(dst.at[pl.ds(e0, CH)], S["dst"], S["seml"])
            pltpu.async_copy(f0h.at[pl.ds(e0, CH)], S["f0"], S["seml"])
            pltpu.async_copy(f1h.at[pl.ds(e0, CH)], S["f1"], S["seml"])
            pltpu.async_copy(uflat.at[pl.ds(e0 * 16, CH * 16)], S["u"], S["seml"])

    def issue_g(jj, S):
        ch = jj * NSUB + s

        @pl.when(ch < NCHUNK)
        def _():
            pltpu.make_async_copy(src.at[pl.ds(0, CH)], S["src"], S["seml"]).wait()
            pltpu.make_async_copy(dst.at[pl.ds(0, CH)], S["dst"], S["seml"]).wait()
            pltpu.make_async_copy(f0h.at[pl.ds(0, CH)], S["f0"], S["seml"]).wait()
            pltpu.make_async_copy(f1h.at[pl.ds(0, CH)], S["f1"], S["seml"]).wait()
            pltpu.make_async_copy(uflat.at[pl.ds(0, CH * 16)], S["u"], S["seml"]).wait()
            for i in range(CH // 16):
                sl = pl.ds(i * 16, 16)
                S["dst"][sl] = S["dst"][sl] + c * N
            pltpu.async_copy(vT.at[S["dst"]], S["vg"], S["semg"])

    def do_c(jj, S):
        ch = jj * NSUB + s

        @pl.when(ch < NCHUNK)
        def _():
            @pl.when(jj >= 2)
            def _():
                pltpu.make_async_copy(S["dv"], acc.at[S["s1i"]], S["sems"]).wait()
            pltpu.make_async_copy(vT.at[S["dst"]], S["vg"], S["semg"]).wait()
            for i in range(CH // 16):
                sl = pl.ds(i * 16, 16)
                S["s1i"][sl] = S["src"][sl]

            @plsc.parallel_loop(0, CH, 1, unroll=4)
            def edge_body(e):
                uv = S["u"][pl.ds(e * 16, 16)]
                u_own = _splat(uv, c)
                for kg in range(FEAT // 16):
                    ksl = pl.ds(kg * 16, 16)
                    S["dv"][e, ksl] = (S["f0"][e, ksl] * u_own
                                       + S["f1"][e, ksl] * S["vg"][e, ksl])

            pltpu.async_copy(S["dv"], acc.at[S["s1i"]], S["sems"], add=True)

    issue_l(0, sets[0])
    issue_l(1, sets[1])
    issue_g(0, sets[0])

    def pair_body(j2, carry):
        for b in range(2):
            jj = 2 * j2 + b
            do_c(jj, sets[b])
            issue_l(jj + 2, sets[b])
            issue_g(jj + 1, sets[1 - b])
        return carry

    npairs = (nround + 1) // 2
    lax.fori_loop(0, npairs, pair_body, 0)
    for b in range(2):
        pltpu.make_async_copy(sets[b]["dv"], acc.at[sets[b]["s1i"]],
                              sets[b]["sems"]).wait()
    plsc.subcore_barrier()
    pltpu.sync_copy(acc.at[pl.ds(n0, ROWS_A)], vout.at[pl.ds(c * N + n0, ROWS_A)])

    @pl.when(s == NSUB - 1)
    def _():
        pltpu.sync_copy(acc.at[pl.ds(NSUB * ROWS_A, TAIL_A)],
                        vout.at[pl.ds(c * N + NSUB * ROWS_A, TAIL_A)])


def _vscatter_a(vT, f0, f1, uflat, src, dst):
    mesh = plsc.VectorSubcoreMesh(core_axis_name="c", subcore_axis_name="s")
    bufset = [
        pltpu.VMEM((CH,), jnp.int32),
        pltpu.VMEM((CH,), jnp.int32),
        pltpu.VMEM((CH,), jnp.int32),
        pltpu.VMEM((CH, FEAT), jnp.float32),
        pltpu.VMEM((CH, FEAT), jnp.float32),
        pltpu.VMEM((CH * 16,), jnp.float32),
        pltpu.VMEM((CH, FEAT), jnp.float32),
        pltpu.VMEM((CH, FEAT), jnp.float32),
        pltpu.SemaphoreType.DMA,
    ]
    k = pl.kernel(
        _vscat_a_body,
        out_type=jax.ShapeDtypeStruct((2 * N, FEAT), jnp.float32),
        mesh=mesh,
        scratch_types=[pltpu.VMEM_SHARED((N, FEAT), jnp.float32)]
        + bufset + bufset + [pltpu.SemaphoreType.DMA] * 4,
    )
    return k(vT, f0, f1, uflat, src, dst)


_B_KEYS = ("src", "dst", "s1i", "f0", "f1", "u", "vg", "dv", "hs", "seml")


def _vscat_b_body(vT, h_i, f0h, f1h, uflat, src, dst, v2out, hout, acc, *bufs):
    c = lax.axis_index("c")
    s = lax.axis_index("s")

    sets = []
    for b in range(2):
        d = dict(zip(_B_KEYS, bufs[b * 10:b * 10 + 10]))
        d["semg"] = bufs[20 + 2 * b]
        d["sems"] = bufs[21 + 2 * b]
        sets.append(d)

    n0 = s * ROWS_A

    @pl.when(c == 0)
    def _():
        pltpu.sync_copy(vT.at[pl.ds(2 * N + n0, ROWS_A)],
                        acc.at[pl.ds(n0, ROWS_A)])

        @pl.when(s == NSUB - 1)
        def _():
            pltpu.sync_copy(vT.at[pl.ds(2 * N + NSUB * ROWS_A, TAIL_A)],
                            acc.at[pl.ds(NSUB * ROWS_A, TAIL_A)])

    @pl.when(c == 1)
    def _():
        pltpu.sync_copy(h_i.at[pl.ds(n0, ROWS_A)], acc.at[pl.ds(n0, ROWS_A)])

        @pl.when(s == NSUB - 1)
        def _():
            pltpu.sync_copy(h_i.at[pl.ds(NSUB * ROWS_A, TAIL_A)],
                            acc.at[pl.ds(NSUB * ROWS_A, TAIL_A)])

    plsc.subcore_barrier()
    nround = (NCHUNK + NSUB - 1) // NSUB

    def issue_l(jj, S):
        ch = jj * NSUB + s

        @pl.when(ch < NCHUNK)
        def _():
            e0 = ch * CH
            pltpu.async_copy(src.at[pl.ds(e0, CH)], S["src"], S["seml"])
            pltpu.async_copy(f1h.at[pl.ds(e0, CH)], S["f1"], S["seml"])

            @pl.when(c == 0)
            def _():
                pltpu.async_copy(dst.at[pl.ds(e0, CH)], S["dst"], S["seml"])
                pltpu.async_copy(f0h.at[pl.ds(e0, CH)], S["f0"], S["seml"])
                pltpu.async_copy(uflat.at[pl.ds(e0 * 16, CH * 16)], S["u"],
                                 S["seml"])

    def issue_g(jj, S):
        ch = jj * NSUB + s

        @pl.when(ch < NCHUNK)
        def _():
            pltpu.make_async_copy(src.at[pl.ds(0, CH)], S["src"], S["seml"]).wait()
            pltpu.make_async_copy(f1h.at[pl.ds(0, CH)], S["f1"], S["seml"]).wait()

            @pl.when(c == 0)
            def _():
                pltpu.make_async_copy(dst.at[pl.ds(0, CH)], S["dst"],
                                      S["seml"]).wait()
                pltpu.make_async_copy(f0h.at[pl.ds(0, CH)], S["f0"],
                                      S["seml"]).wait()
                pltpu.make_async_copy(uflat.at[pl.ds(0, CH * 16)], S["u"],
                                      S["seml"]).wait()
                for i in range(CH // 16):
                    sl = pl.ds(i * 16, 16)
                    S["dst"][sl] = S["dst"][sl] + 2 * N
                pltpu.async_copy(vT.at[S["dst"]], S["vg"], S["semg"])

    def do_c(jj, S):
        ch = jj * NSUB + s

        @pl.when(ch < NCHUNK)
        def _():
            @pl.when(jj >= 2)
            def _():
                pltpu.make_async_copy(S["dv"], acc.at[S["s1i"]], S["sems"]).wait()
            for i in range(CH // 16):
                sl = pl.ds(i * 16, 16)
                S["s1i"][sl] = S["src"][sl]

            @pl.when(c == 0)
            def _():
                pltpu.make_async_copy(vT.at[S["dst"]], S["vg"], S["semg"]).wait()

                @plsc.parallel_loop(0, CH, 1, unroll=4)
                def edge_body(e):
                    uv = S["u"][pl.ds(e * 16, 16)]
                    u_2 = _splat(uv, 2)
                    for kg in range(FEAT // 16):
                        ksl = pl.ds(kg * 16, 16)
                        S["dv"][e, ksl] = (S["f0"][e, ksl] * u_2
                                           + S["f1"][e, ksl] * S["vg"][e, ksl])

                pltpu.async_copy(S["dv"], acc.at[S["s1i"]], S["sems"], add=True)

            @pl.when(c == 1)
            def _():
                @plsc.parallel_loop(0, CH, 1, unroll=4)
                def edge_body(e):
                    for kg in range(FEAT // 16):
                        ksl = pl.ds(kg * 16, 16)
                        S["hs"][e, ksl] = S["f1"][e, ksl]

                pltpu.async_copy(S["hs"], acc.at[S["s1i"]], S["sems"], add=True)

    issue_l(0, sets[0])
    issue_l(1, sets[1])
    issue_g(0, sets[0])

    def pair_body(j2, carry):
        for b in range(2):
            jj = 2 * j2 + b
            do_c(jj, sets[b])
            issue_l(jj + 2, sets[b])
            issue_g(jj + 1, sets[1 - b])
        return carry

    npairs = (nround + 1) // 2
    lax.fori_loop(0, npairs, pair_body, 0)
    for b in range(2):
        pltpu.make_async_copy(sets[b]["dv"], acc.at[sets[b]["s1i"]],
                              sets[b]["sems"]).wait()
    plsc.subcore_barrier()

    @pl.when(c == 0)
    def _():
        pltpu.sync_copy(acc.at[pl.ds(n0, ROWS_A)], v2out.at[pl.ds(n0, ROWS_A)])

        @pl.when(s == NSUB - 1)
        def _():
            pltpu.sync_copy(acc.at[pl.ds(NSUB * ROWS_A, TAIL_A)],
                            v2out.at[pl.ds(NSUB * ROWS_A, TAIL_A)])

    @pl.when(c == 1)
    def _():
        pltpu.sync_copy(acc.at[pl.ds(n0, ROWS_A)], hout.at[pl.ds(n0, ROWS_A)])

        @pl.when(s == NSUB - 1)
        def _():
            pltpu.sync_copy(acc.at[pl.ds(NSUB * ROWS_A, TAIL_A)],
                            hout.at[pl.ds(NSUB * ROWS_A, TAIL_A)])


def _vscatter_b(vT, h_i, f0, f1, uflat, src, dst):
    mesh = plsc.VectorSubcoreMesh(core_axis_name="c", subcore_axis_name="s")
    bufset = [
        pltpu.VMEM((CH,), jnp.int32),
        pltpu.VMEM((CH,), jnp.int32),
        pltpu.VMEM((CH,), jnp.int32),
        pltpu.VMEM((CH, FEAT), jnp.float32),
        pltpu.VMEM((CH, FEAT), jnp.float32),
        pltpu.VMEM((CH * 16,), jnp.float32),
        pltpu.VMEM((CH, FEAT), jnp.float32),
        pltpu.VMEM((CH, FEAT), jnp.float32),
        pltpu.VMEM((CH, FEAT), jnp.float32),
        pltpu.SemaphoreType.DMA,
    ]
    k = pl.kernel(
        _vscat_b_body,
        out_type=[jax.ShapeDtypeStruct((N, FEAT), jnp.float32),
                  jax.ShapeDtypeStruct((N, FEAT), jnp.float32)],
        mesh=mesh,
        scratch_types=[pltpu.VMEM_SHARED((N, FEAT), jnp.float32)]
        + bufset + bufset + [pltpu.SemaphoreType.DMA] * 4,
    )
    return k(vT, h_i, f0, f1, uflat, src, dst)


def kernel(h_i, v_i, d_ij, unit_r_ij, nbrs, W1, b1, W2, b2,
           Wf0, bf0, Wf1, bf1, Wf2, bf2, We1, be1, We2, be2):
    src = nbrs[:, 0]
    dst = nbrs[:, 1]

    phi = _phi(h_i, W1, b1, W2, b2)
    G = _gather_phi(phi, src)
    f0, f1 = _edge_filters(d_ij, G, We1, be1, We2, be2, Wf0, bf0, Wf1, bf1)

    vT = v_i.transpose(2, 0, 1).reshape(3 * N, FEAT)
    uflat = jnp.pad(unit_r_ij, ((0, 0), (0, 13))).reshape(16 * E)

    vout01 = _vscatter_a(vT, f0, f1, uflat, src, dst)
    vout2, h_out = _vscatter_b(vT, h_i, f0, f1, uflat, src, dst)

    v_out = jnp.stack([vout01[:N], vout01[N:], vout2], axis=0).transpose(1, 2, 0)
    return (h_out, v_out)
